# Initial kernel scaffold; baseline (speedup 1.0000x reference)
import jax
import jax.numpy as jnp
from jax.experimental import pallas as pl


def kernel(x, w):
    raise NotImplementedError("write your pallas kernel here")


# speedup vs baseline: 1.4581x; 1.4581x over previous
import functools

import jax
import jax.numpy as jnp
from jax import lax
from jax.experimental import pallas as pl
from jax.experimental.pallas import tpu as pltpu
from jax.experimental.pallas import tpu_sc as plsc

_D = 32
_RPS = 128
_NC = 2
_NS = 16
_NW = _NC * _NS


@functools.lru_cache(maxsize=None)
def _gather_call(tot, kk, ng):
    ch = kk * _RPS
    mesh = plsc.VectorSubcoreMesh(core_axis_name="c", subcore_axis_name="s")

    @functools.partial(
        pl.kernel,
        mesh=mesh,
        out_type=jax.ShapeDtypeStruct((tot, _D), jnp.float32),
        scratch_types=[
            pltpu.VMEM((kk, _RPS), jnp.int32),
            pltpu.VMEM((ch, _D), jnp.float32),
            pltpu.SemaphoreType.DMA,
        ],
        compiler_params=pltpu.CompilerParams(use_tc_tiling_on_sc=False),
    )
    def k(idx_hbm, tab_hbm, out_hbm, idx_v, rows_v, sem):
        wid = lax.axis_index("s") * _NC + lax.axis_index("c")
        row0 = wid * (ng * kk)

        def body(gi, carry):
            irow = row0 + gi * kk
            pltpu.sync_copy(idx_hbm.at[pl.ds(irow, kk)], idx_v)
            copies = [
                pltpu.async_copy(
                    tab_hbm.at[idx_v.at[j]],
                    rows_v.at[pl.ds(j * _RPS, _RPS)],
                    sem,
                )
                for j in range(kk)
            ]
            for c in copies:
                c.wait()
            pltpu.sync_copy(rows_v, out_hbm.at[pl.ds(irow * _RPS, ch)])
            return carry

        lax.fori_loop(0, ng, body, 0)

    return k


def kernel(x, w):
    b, s = x.shape
    tot = b * s
    kk = 8
    assert tot % (_NW * kk * _RPS) == 0
    ng = tot // (_NW * kk * _RPS)
    idx = x.astype(jnp.int32).reshape(tot // _RPS, _RPS)
    out = _gather_call(tot, kk, ng)(idx, w)
    return out.reshape(b, s, _D)


# speedup vs baseline: 1.4941x; 1.0247x over previous
import functools

import jax
import jax.numpy as jnp
from jax import lax
from jax.experimental import pallas as pl
from jax.experimental.pallas import tpu as pltpu
from jax.experimental.pallas import tpu_sc as plsc

_D = 32
_RPS = 128
_NC = 2
_NS = 16
_NW = _NC * _NS


@functools.lru_cache(maxsize=None)
def _gather_call(tot, kk, ng):
    ch = kk * _RPS
    ipw = ng * kk
    mesh = plsc.VectorSubcoreMesh(core_axis_name="c", subcore_axis_name="s")

    @functools.partial(
        pl.kernel,
        mesh=mesh,
        out_type=jax.ShapeDtypeStruct((tot, _D), jnp.float32),
        scratch_types=[
            pltpu.VMEM((ipw, _RPS), jnp.int32),
            pltpu.VMEM((2, ch, _D), jnp.float32),
            pltpu.SemaphoreType.DMA,
            (pltpu.SemaphoreType.DMA, pltpu.SemaphoreType.DMA),
        ],
        compiler_params=pltpu.CompilerParams(use_tc_tiling_on_sc=False),
    )
    def k(idx_hbm, tab_hbm, out_hbm, idx_v, rows_v, gsem, wsems):
        wid = lax.axis_index("s") * _NC + lax.axis_index("c")
        row0 = wid * ipw
        pltpu.sync_copy(idx_hbm.at[pl.ds(row0, ipw)], idx_v)

        def wb_wait(b):
            pltpu.make_async_copy(
                rows_v.at[b], out_hbm.at[pl.ds(0, ch)], wsems[b]).wait()

        def do_chunk(g, b):
            irow = g * kk
            copies = [
                pltpu.async_copy(
                    tab_hbm.at[idx_v.at[irow + j]],
                    rows_v.at[b, pl.ds(j * _RPS, _RPS)],
                    gsem,
                )
                for j in range(kk)
            ]
            for c in copies:
                c.wait()
            pltpu.make_async_copy(
                rows_v.at[b],
                out_hbm.at[pl.ds((row0 + irow) * _RPS, ch)],
                wsems[b],
            ).start()

        def body(g2, carry):
            g = g2 * 2

            @pl.when(g2 > 0)
            def _():
                wb_wait(0)

            do_chunk(g, 0)

            @pl.when(g2 > 0)
            def _():
                wb_wait(1)

            do_chunk(g + 1, 1)
            return carry

        lax.fori_loop(0, ng // 2, body, 0)
        wb_wait(0)
        wb_wait(1)

    return k


def kernel(x, w):
    b, s = x.shape
    tot = b * s
    kk = 10
    assert tot % (_NW * 2 * kk * _RPS) == 0
    ng = tot // (_NW * kk * _RPS)
    idx = x.astype(jnp.int32).reshape(tot // _RPS, _RPS)
    out = _gather_call(tot, kk, ng)(idx, w)
    return out.reshape(b, s, _D)
